# baseline (speedup 1.0000x reference)
import functools
import jax
import jax.numpy as jnp
from jax import lax
from jax.experimental import pallas as pl
from jax.experimental.pallas import tpu as pltpu
from jax.experimental.pallas import tpu_sc as plsc

M = 16384
N = 16384

BM = 256
LANES = 128


def _mv_block(a_ref, x_ref, o_ref):
    a = a_ref[...]
    x = x_ref[...]
    acc = jnp.zeros((BM, LANES), jnp.float32)
    for k in range(N // LANES):
        sl = slice(k * LANES, (k + 1) * LANES)
        acc = acc + a[:, sl] * x[:, sl]
    o_ref[...] = jnp.sum(acc, axis=1, keepdims=True)


def _tc_mv(A, x):
    out = pl.pallas_call(
        _mv_block,
        grid=(M // BM,),
        in_specs=[
            pl.BlockSpec((BM, N), lambda i: (i, 0)),
            pl.BlockSpec((1, N), lambda i: (0, 0)),
        ],
        out_specs=pl.BlockSpec((BM, 1), lambda i: (i, 0)),
        out_shape=jax.ShapeDtypeStruct((M, 1), jnp.float32),
    )(A, x.reshape(1, N))
    return out.reshape(M)


NC = 2
NS = 16
NW = NC * NS
ROWS_W = M // NW
R = 16
NPASS = 8
QCOL = N // NPASS
QPAD = QCOL + 1
NCHUNK = ROWS_W // R

_mesh = plsc.VectorSubcoreMesh(core_axis_name="c", subcore_axis_name="s")


@functools.partial(
    pl.kernel,
    out_type=jax.ShapeDtypeStruct((M,), jnp.float32),
    mesh=_mesh,
    scratch_types=[
        pltpu.VMEM((QCOL,), jnp.float32),
        pltpu.VMEM((R, QPAD), jnp.float32),
        pltpu.VMEM((R, QPAD), jnp.float32),
        pltpu.VMEM((ROWS_W,), jnp.float32),
        pltpu.SemaphoreType.DMA,
        pltpu.SemaphoreType.DMA,
    ],
    compiler_params=pltpu.CompilerParams(use_tc_tiling_on_sc=False,
                                         needs_layout_passes=False),
)
def _sc_mv(a_hbm, x_hbm, o_hbm, x_v, buf0, buf1, out_v, sem0, sem1):
    wid = lax.axis_index("s") * NC + lax.axis_index("c")
    row0 = wid * ROWS_W
    bufs = (buf0, buf1)
    sems = (sem0, sem1)
    lanes = lax.iota(jnp.int32, 16)

    for p in range(NPASS):
        col0 = p * QCOL
        pltpu.sync_copy(x_hbm.at[pl.ds(col0, QCOL)], x_v)
        for b in range(2):
            pltpu.make_async_copy(
                a_hbm.at[pl.ds(row0 + b * R, R), pl.ds(col0, QCOL)],
                bufs[b].at[:, pl.ds(0, QCOL)], sems[b]).start()

        def chunk_pair(g, carry, p=p, col0=col0):
            for b in range(2):
                c = g * 2 + b
                pltpu.make_async_copy(
                    a_hbm.at[pl.ds(row0 + c * R, R), pl.ds(col0, QCOL)],
                    bufs[b].at[:, pl.ds(0, QCOL)], sems[b]).wait()

                def kbody(j, acc, b=b):
                    xv = x_v[pl.ds(j * 16, 16)]
                    base = j * 16
                    for t in range(16):
                        col = jnp.full((16,), base + t, jnp.int32)
                        g16 = plsc.load_gather(bufs[b], [lanes, col])
                        acc = acc + g16 * xv[t]
                    return acc
                acc = lax.fori_loop(0, QCOL // 16, kbody,
                                    jnp.zeros((16,), jnp.float32),
                                    unroll=2)

                @pl.when(c + 2 < NCHUNK)
                def _(b=b, c=c):
                    pltpu.make_async_copy(
                        a_hbm.at[pl.ds(row0 + (c + 2) * R, R),
                                 pl.ds(col0, QCOL)],
                        bufs[b].at[:, pl.ds(0, QCOL)], sems[b]).start()

                sl = pl.ds(c * 16, 16)
                if p == 0:
                    out_v[sl] = acc
                else:
                    out_v[sl] = out_v[sl] + acc
            return carry

        lax.fori_loop(0, NCHUNK // 2, chunk_pair, 0)

    pltpu.sync_copy(out_v, o_hbm.at[pl.ds(row0, ROWS_W)])


@jax.jit
def _mv(A, x):
    return _sc_mv(A, x)


def kernel(A, x):
    return _mv(A, x)


# speedup vs baseline: 1.0000x; 1.0000x over previous
import functools
import jax
import jax.numpy as jnp
from jax import lax
from jax.experimental import pallas as pl
from jax.experimental.pallas import tpu as pltpu
from jax.experimental.pallas import tpu_sc as plsc

M = 16384
N = 16384

BM = 256
LANES = 128


def _mv_block(a_ref, x_ref, o_ref):
    a = a_ref[...]
    x = x_ref[...]
    acc = jnp.zeros((BM, LANES), jnp.float32)
    for k in range(N // LANES):
        sl = slice(k * LANES, (k + 1) * LANES)
        acc = acc + a[:, sl] * x[:, sl]
    o_ref[...] = jnp.sum(acc, axis=1, keepdims=True)


def _tc_mv(A, x):
    out = pl.pallas_call(
        _mv_block,
        grid=(M // BM,),
        in_specs=[
            pl.BlockSpec((BM, N), lambda i: (i, 0)),
            pl.BlockSpec((1, N), lambda i: (0, 0)),
        ],
        out_specs=pl.BlockSpec((BM, 1), lambda i: (i, 0)),
        out_shape=jax.ShapeDtypeStruct((M, 1), jnp.float32),
    )(A, x.reshape(1, N))
    return out.reshape(M)


NC = 2
NS = 16
NW = NC * NS
ROWS_W = M // NW
R = 16
NPASS = 8
QCOL = N // NPASS
QPAD = QCOL + 8
NCHUNK = ROWS_W // R

_mesh = plsc.VectorSubcoreMesh(core_axis_name="c", subcore_axis_name="s")


@functools.partial(
    pl.kernel,
    out_type=jax.ShapeDtypeStruct((M,), jnp.float32),
    mesh=_mesh,
    scratch_types=[
        pltpu.VMEM((QCOL,), jnp.float32),
        pltpu.VMEM((R, QPAD), jnp.float32),
        pltpu.VMEM((R, QPAD), jnp.float32),
        pltpu.VMEM((ROWS_W,), jnp.float32),
        pltpu.SemaphoreType.DMA,
        pltpu.SemaphoreType.DMA,
    ],
    compiler_params=pltpu.CompilerParams(use_tc_tiling_on_sc=False,
                                         needs_layout_passes=False),
)
def _sc_mv(a_hbm, x_hbm, o_hbm, x_v, buf0, buf1, out_v, sem0, sem1):
    wid = lax.axis_index("s") * NC + lax.axis_index("c")
    row0 = wid * ROWS_W
    bufs = (buf0, buf1)
    sems = (sem0, sem1)
    lanes = lax.iota(jnp.int32, 16)

    for p in range(NPASS):
        col0 = p * QCOL
        pltpu.sync_copy(x_hbm.at[pl.ds(col0, QCOL)], x_v)
        for b in range(2):
            pltpu.make_async_copy(
                a_hbm.at[pl.ds(row0 + b * R, R), pl.ds(col0, QCOL)],
                bufs[b].at[:, pl.ds(0, QCOL)], sems[b]).start()

        def chunk_pair(g, carry, p=p, col0=col0):
            for b in range(2):
                c = g * 2 + b
                pltpu.make_async_copy(
                    a_hbm.at[pl.ds(row0 + c * R, R), pl.ds(col0, QCOL)],
                    bufs[b].at[:, pl.ds(0, QCOL)], sems[b]).wait()

                def kbody(j, acc, b=b):
                    xv = x_v[pl.ds(j * 16, 16)]
                    base = j * 16
                    for t in range(16):
                        col = jnp.full((16,), base + t, jnp.int32)
                        g16 = plsc.load_gather(bufs[b], [lanes, col])
                        acc = acc + g16 * xv[t]
                    return acc
                acc = lax.fori_loop(0, QCOL // 16, kbody,
                                    jnp.zeros((16,), jnp.float32),
                                    unroll=2)

                @pl.when(c + 2 < NCHUNK)
                def _(b=b, c=c):
                    pltpu.make_async_copy(
                        a_hbm.at[pl.ds(row0 + (c + 2) * R, R),
                                 pl.ds(col0, QCOL)],
                        bufs[b].at[:, pl.ds(0, QCOL)], sems[b]).start()

                sl = pl.ds(c * 16, 16)
                if p == 0:
                    out_v[sl] = acc
                else:
                    out_v[sl] = out_v[sl] + acc
            return carry

        lax.fori_loop(0, NCHUNK // 2, chunk_pair, 0)

    pltpu.sync_copy(out_v, o_hbm.at[pl.ds(row0, ROWS_W)])


@jax.jit
def _mv(A, x):
    return _sc_mv(A, x)


def kernel(A, x):
    return _mv(A, x)


# speedup vs baseline: 3.1295x; 3.1295x over previous
import functools
import jax
import jax.numpy as jnp
from jax import lax
from jax.experimental import pallas as pl
from jax.experimental.pallas import tpu as pltpu
from jax.experimental.pallas import tpu_sc as plsc

M = 16384
N = 16384

BM = 256
LANES = 128


def _mv_block(a_ref, x_ref, o_ref):
    a = a_ref[...]
    x = x_ref[...]
    acc = jnp.zeros((BM, LANES), jnp.float32)
    for k in range(N // LANES):
        sl = slice(k * LANES, (k + 1) * LANES)
        acc = acc + a[:, sl] * x[:, sl]
    o_ref[...] = jnp.sum(acc, axis=1, keepdims=True)


def _tc_mv(A, x):
    out = pl.pallas_call(
        _mv_block,
        grid=(M // BM,),
        in_specs=[
            pl.BlockSpec((BM, N), lambda i: (i, 0)),
            pl.BlockSpec((1, N), lambda i: (0, 0)),
        ],
        out_specs=pl.BlockSpec((BM, 1), lambda i: (i, 0)),
        out_shape=jax.ShapeDtypeStruct((M, 1), jnp.float32),
    )(A, x.reshape(1, N))
    return out.reshape(M)


NC = 2
NS = 16
NW = NC * NS
ROWS_W = M // NW
R = 8
NPASS = 8
QCOL = N // NPASS
NCHUNK = ROWS_W // R
KI = QCOL // 16

_mesh = plsc.VectorSubcoreMesh(core_axis_name="c", subcore_axis_name="s")


@functools.partial(
    pl.kernel,
    out_type=jax.ShapeDtypeStruct((M,), jnp.float32),
    mesh=_mesh,
    scratch_types=[
        pltpu.VMEM((QCOL,), jnp.float32),
        pltpu.VMEM((R, QCOL), jnp.float32),
        pltpu.VMEM((R, QCOL), jnp.float32),
        pltpu.VMEM((ROWS_W,), jnp.float32),
        pltpu.SemaphoreType.DMA,
        pltpu.SemaphoreType.DMA,
    ],
    compiler_params=pltpu.CompilerParams(needs_layout_passes=False),
)
def _sc_mv(a_hbm, x_hbm, o_hbm, x_v, buf0, buf1, out_v, sem0, sem1):
    wid = lax.axis_index("s") * NC + lax.axis_index("c")
    row0 = wid * ROWS_W
    bufs = (buf0, buf1)
    sems = (sem0, sem1)
    lanes = lax.iota(jnp.int32, 16)

    for p in range(NPASS):
        col0 = p * QCOL
        pltpu.sync_copy(x_hbm.at[pl.ds(col0, QCOL)], x_v)
        for b in range(2):
            pltpu.make_async_copy(
                a_hbm.at[pl.ds(row0 + b * R, R), pl.ds(col0, QCOL)],
                bufs[b], sems[b]).start()

        def chunk_pair(g, carry, p=p, col0=col0):
            resv = jnp.zeros((16,), jnp.float32)
            for b in range(2):
                c = g * 2 + b
                pltpu.make_async_copy(
                    a_hbm.at[pl.ds(row0 + c * R, R), pl.ds(col0, QCOL)],
                    bufs[b], sems[b]).wait()

                def kbody(k, accs, b=b):
                    xa = x_v[pl.ds(k * 16, 16)]
                    return tuple(accs[r] + bufs[b][r, pl.ds(k * 16, 16)] * xa
                                 for r in range(R))
                accs = lax.fori_loop(
                    0, KI, kbody,
                    tuple(jnp.zeros((16,), jnp.float32) for _ in range(R)))

                @pl.when(c + 2 < NCHUNK)
                def _(b=b, c=c):
                    pltpu.make_async_copy(
                        a_hbm.at[pl.ds(row0 + (c + 2) * R, R),
                                 pl.ds(col0, QCOL)],
                        bufs[b], sems[b]).start()

                for r in range(R):
                    s = jnp.sum(accs[r])
                    resv = jnp.where(lanes == (b * R + r), s, resv)
            sl = pl.ds(g * 16, 16)
            if p == 0:
                out_v[sl] = resv
            else:
                out_v[sl] = out_v[sl] + resv
            return carry

        lax.fori_loop(0, NCHUNK // 2, chunk_pair, 0)

    pltpu.sync_copy(out_v, o_hbm.at[pl.ds(row0, ROWS_W)])


@jax.jit
def _mv(A, x):
    return _sc_mv(A, x)


def kernel(A, x):
    return _mv(A, x)


# speedup vs baseline: 4.9824x; 1.5921x over previous
import functools
import jax
import jax.numpy as jnp
from jax import lax
from jax.experimental import pallas as pl
from jax.experimental.pallas import tpu as pltpu
from jax.experimental.pallas import tpu_sc as plsc

M = 16384
N = 16384

BM = 256
LANES = 128


def _mv_block(a_ref, x_ref, o_ref):
    a = a_ref[...]
    x = x_ref[...]
    acc = jnp.zeros((BM, LANES), jnp.float32)
    for k in range(N // LANES):
        sl = slice(k * LANES, (k + 1) * LANES)
        acc = acc + a[:, sl] * x[:, sl]
    o_ref[...] = jnp.sum(acc, axis=1, keepdims=True)


def _tc_mv(A, x, row_base, rows):
    base_blk = row_base // BM
    out = pl.pallas_call(
        _mv_block,
        grid=(rows // BM,),
        in_specs=[
            pl.BlockSpec((BM, N), lambda i: (i + base_blk, 0)),
            pl.BlockSpec((1, N), lambda i: (0, 0)),
        ],
        out_specs=pl.BlockSpec((BM, 1), lambda i: (i, 0)),
        out_shape=jax.ShapeDtypeStruct((rows, 1), jnp.float32),
    )(A, x.reshape(1, N))
    return out.reshape(rows)


NC = 2
NS = 16
NW = NC * NS
ROWS_SC = 6144
ROWS_W = ROWS_SC // NW
R = 8
NPASS = 8
QCOL = N // NPASS
NCHUNK = ROWS_W // R
KI = QCOL // 16

_mesh = plsc.VectorSubcoreMesh(core_axis_name="c", subcore_axis_name="s")


@functools.partial(
    pl.kernel,
    out_type=jax.ShapeDtypeStruct((ROWS_SC,), jnp.float32),
    mesh=_mesh,
    scratch_types=[
        pltpu.VMEM((QCOL,), jnp.float32),
        pltpu.VMEM((R, QCOL), jnp.float32),
        pltpu.VMEM((R, QCOL), jnp.float32),
        pltpu.VMEM((ROWS_W,), jnp.float32),
        pltpu.SemaphoreType.DMA,
        pltpu.SemaphoreType.DMA,
    ],
    compiler_params=pltpu.CompilerParams(needs_layout_passes=False),
)
def _sc_mv(a_hbm, x_hbm, o_hbm, x_v, buf0, buf1, out_v, sem0, sem1):
    wid = lax.axis_index("s") * NC + lax.axis_index("c")
    row0 = wid * ROWS_W
    bufs = (buf0, buf1)
    sems = (sem0, sem1)
    lanes = lax.iota(jnp.int32, 16)

    for p in range(NPASS):
        col0 = p * QCOL
        pltpu.sync_copy(x_hbm.at[pl.ds(col0, QCOL)], x_v)
        for b in range(2):
            pltpu.make_async_copy(
                a_hbm.at[pl.ds(row0 + b * R, R), pl.ds(col0, QCOL)],
                bufs[b], sems[b]).start()

        def chunk_pair(g, carry, p=p, col0=col0):
            resv = jnp.zeros((16,), jnp.float32)
            for b in range(2):
                c = g * 2 + b
                pltpu.make_async_copy(
                    a_hbm.at[pl.ds(row0 + c * R, R), pl.ds(col0, QCOL)],
                    bufs[b], sems[b]).wait()

                def kbody(k, accs, b=b):
                    xa = x_v[pl.ds(k * 16, 16)]
                    return tuple(accs[r] + bufs[b][r, pl.ds(k * 16, 16)] * xa
                                 for r in range(R))
                accs = lax.fori_loop(
                    0, KI, kbody,
                    tuple(jnp.zeros((16,), jnp.float32) for _ in range(R)))

                @pl.when(c + 2 < NCHUNK)
                def _(b=b, c=c):
                    pltpu.make_async_copy(
                        a_hbm.at[pl.ds(row0 + (c + 2) * R, R),
                                 pl.ds(col0, QCOL)],
                        bufs[b], sems[b]).start()

                for r in range(R):
                    s = jnp.sum(accs[r])
                    resv = jnp.where(lanes == (b * R + r), s, resv)
            sl = pl.ds(g * 16, 16)
            if p == 0:
                out_v[sl] = resv
            else:
                out_v[sl] = out_v[sl] + resv
            return carry

        lax.fori_loop(0, NCHUNK // 2, chunk_pair, 0)

    pltpu.sync_copy(out_v, o_hbm.at[pl.ds(row0, ROWS_W)])


@jax.jit
def _mv(A, x):
    o_sc = _sc_mv(A, x)
    o_tc = _tc_mv(A, x, ROWS_SC, M - ROWS_SC)
    return jnp.concatenate([o_sc, o_tc])


def kernel(A, x):
    return _mv(A, x)


# speedup vs baseline: 5.0598x; 1.0155x over previous
import functools
import jax
import jax.numpy as jnp
from jax import lax
from jax.experimental import pallas as pl
from jax.experimental.pallas import tpu as pltpu
from jax.experimental.pallas import tpu_sc as plsc

M = 16384
N = 16384

BM = 256
LANES = 128


def _mv_block(a_ref, x_ref, o_ref):
    a = a_ref[...]
    x = x_ref[...]
    acc = jnp.zeros((BM, LANES), jnp.float32)
    for k in range(N // LANES):
        sl = slice(k * LANES, (k + 1) * LANES)
        acc = acc + a[:, sl] * x[:, sl]
    o_ref[...] = jnp.sum(acc, axis=1, keepdims=True)


def _tc_mv(A, x, row_base, rows):
    base_blk = row_base // BM
    out = pl.pallas_call(
        _mv_block,
        grid=(rows // BM,),
        in_specs=[
            pl.BlockSpec((BM, N), lambda i: (i + base_blk, 0)),
            pl.BlockSpec((1, N), lambda i: (0, 0)),
        ],
        out_specs=pl.BlockSpec((BM, 1), lambda i: (i, 0)),
        out_shape=jax.ShapeDtypeStruct((rows, 1), jnp.float32),
    )(A, x.reshape(1, N))
    return out.reshape(rows)


NC = 2
NS = 16
NW = NC * NS
ROWS_SC = 2048
ROWS_W = ROWS_SC // NW
R = 8
NPASS = 8
QCOL = N // NPASS
NCHUNK = ROWS_W // R
KI = QCOL // 16

_mesh = plsc.VectorSubcoreMesh(core_axis_name="c", subcore_axis_name="s")


@functools.partial(
    pl.kernel,
    out_type=jax.ShapeDtypeStruct((ROWS_SC,), jnp.float32),
    mesh=_mesh,
    scratch_types=[
        pltpu.VMEM((QCOL,), jnp.float32),
        pltpu.VMEM((R, QCOL), jnp.float32),
        pltpu.VMEM((R, QCOL), jnp.float32),
        pltpu.VMEM((ROWS_W,), jnp.float32),
        pltpu.SemaphoreType.DMA,
        pltpu.SemaphoreType.DMA,
    ],
    compiler_params=pltpu.CompilerParams(needs_layout_passes=False),
)
def _sc_mv(a_hbm, x_hbm, o_hbm, x_v, buf0, buf1, out_v, sem0, sem1):
    wid = lax.axis_index("s") * NC + lax.axis_index("c")
    row0 = wid * ROWS_W
    bufs = (buf0, buf1)
    sems = (sem0, sem1)
    lanes = lax.iota(jnp.int32, 16)

    for p in range(NPASS):
        col0 = p * QCOL
        pltpu.sync_copy(x_hbm.at[pl.ds(col0, QCOL)], x_v)
        for b in range(2):
            pltpu.make_async_copy(
                a_hbm.at[pl.ds(row0 + b * R, R), pl.ds(col0, QCOL)],
                bufs[b], sems[b]).start()

        def chunk_pair(g, carry, p=p, col0=col0):
            resv = jnp.zeros((16,), jnp.float32)
            for b in range(2):
                c = g * 2 + b
                pltpu.make_async_copy(
                    a_hbm.at[pl.ds(row0 + c * R, R), pl.ds(col0, QCOL)],
                    bufs[b], sems[b]).wait()

                def kbody(k, accs, b=b):
                    xa = x_v[pl.ds(k * 16, 16)]
                    return tuple(accs[r] + bufs[b][r, pl.ds(k * 16, 16)] * xa
                                 for r in range(R))
                accs = lax.fori_loop(
                    0, KI, kbody,
                    tuple(jnp.zeros((16,), jnp.float32) for _ in range(R)))

                @pl.when(c + 2 < NCHUNK)
                def _(b=b, c=c):
                    pltpu.make_async_copy(
                        a_hbm.at[pl.ds(row0 + (c + 2) * R, R),
                                 pl.ds(col0, QCOL)],
                        bufs[b], sems[b]).start()

                for r in range(R):
                    s = jnp.sum(accs[r])
                    resv = jnp.where(lanes == (b * R + r), s, resv)
            sl = pl.ds(g * 16, 16)
            if p == 0:
                out_v[sl] = resv
            else:
                out_v[sl] = out_v[sl] + resv
            return carry

        lax.fori_loop(0, NCHUNK // 2, chunk_pair, 0)

    pltpu.sync_copy(out_v, o_hbm.at[pl.ds(row0, ROWS_W)])


@jax.jit
def _mv(A, x):
    o_sc = _sc_mv(A, x)
    o_tc = _tc_mv(A, x, ROWS_SC, M - ROWS_SC)
    return jnp.concatenate([o_sc, o_tc])


def kernel(A, x):
    return _mv(A, x)


# speedup vs baseline: 5.0782x; 1.0036x over previous
import functools
import jax
import jax.numpy as jnp
from jax import lax
from jax.experimental import pallas as pl
from jax.experimental.pallas import tpu as pltpu
from jax.experimental.pallas import tpu_sc as plsc

M = 16384
N = 16384

BM = 256
LANES = 128


def _mv_block(a_ref, x_ref, o_ref):
    a = a_ref[...]
    x = x_ref[...]
    acc = jnp.zeros((BM, LANES), jnp.float32)
    for k in range(N // LANES):
        sl = slice(k * LANES, (k + 1) * LANES)
        acc = acc + a[:, sl] * x[:, sl]
    o_ref[...] = jnp.sum(acc, axis=1, keepdims=True)


def _tc_mv(A, x, row_base, rows):
    base_blk = row_base // BM
    out = pl.pallas_call(
        _mv_block,
        grid=(rows // BM,),
        in_specs=[
            pl.BlockSpec((BM, N), lambda i: (i + base_blk, 0)),
            pl.BlockSpec((1, N), lambda i: (0, 0)),
        ],
        out_specs=pl.BlockSpec((BM, 1), lambda i: (i, 0)),
        out_shape=jax.ShapeDtypeStruct((rows, 1), jnp.float32),
    )(A, x.reshape(1, N))
    return out.reshape(rows)


NC = 2
NS = 16
NW = NC * NS
ROWS_SC = 1024
ROWS_W = ROWS_SC // NW
R = 8
NPASS = 8
QCOL = N // NPASS
NCHUNK = ROWS_W // R
KI = QCOL // 16

_mesh = plsc.VectorSubcoreMesh(core_axis_name="c", subcore_axis_name="s")


@functools.partial(
    pl.kernel,
    out_type=jax.ShapeDtypeStruct((ROWS_SC,), jnp.float32),
    mesh=_mesh,
    scratch_types=[
        pltpu.VMEM((QCOL,), jnp.float32),
        pltpu.VMEM((R, QCOL), jnp.float32),
        pltpu.VMEM((R, QCOL), jnp.float32),
        pltpu.VMEM((ROWS_W,), jnp.float32),
        pltpu.SemaphoreType.DMA,
        pltpu.SemaphoreType.DMA,
    ],
    compiler_params=pltpu.CompilerParams(needs_layout_passes=False),
)
def _sc_mv(a_hbm, x_hbm, o_hbm, x_v, buf0, buf1, out_v, sem0, sem1):
    wid = lax.axis_index("s") * NC + lax.axis_index("c")
    row0 = wid * ROWS_W
    bufs = (buf0, buf1)
    sems = (sem0, sem1)
    lanes = lax.iota(jnp.int32, 16)

    for p in range(NPASS):
        col0 = p * QCOL
        pltpu.sync_copy(x_hbm.at[pl.ds(col0, QCOL)], x_v)
        for b in range(2):
            pltpu.make_async_copy(
                a_hbm.at[pl.ds(row0 + b * R, R), pl.ds(col0, QCOL)],
                bufs[b], sems[b]).start()

        def chunk_pair(g, carry, p=p, col0=col0):
            resv = jnp.zeros((16,), jnp.float32)
            for b in range(2):
                c = g * 2 + b
                pltpu.make_async_copy(
                    a_hbm.at[pl.ds(row0 + c * R, R), pl.ds(col0, QCOL)],
                    bufs[b], sems[b]).wait()

                def kbody(k, accs, b=b):
                    xa = x_v[pl.ds(k * 16, 16)]
                    return tuple(accs[r] + bufs[b][r, pl.ds(k * 16, 16)] * xa
                                 for r in range(R))
                accs = lax.fori_loop(
                    0, KI, kbody,
                    tuple(jnp.zeros((16,), jnp.float32) for _ in range(R)))

                @pl.when(c + 2 < NCHUNK)
                def _(b=b, c=c):
                    pltpu.make_async_copy(
                        a_hbm.at[pl.ds(row0 + (c + 2) * R, R),
                                 pl.ds(col0, QCOL)],
                        bufs[b], sems[b]).start()

                for r in range(R):
                    s = jnp.sum(accs[r])
                    resv = jnp.where(lanes == (b * R + r), s, resv)
            sl = pl.ds(g * 16, 16)
            if p == 0:
                out_v[sl] = resv
            else:
                out_v[sl] = out_v[sl] + resv
            return carry

        lax.fori_loop(0, NCHUNK // 2, chunk_pair, 0)

    pltpu.sync_copy(out_v, o_hbm.at[pl.ds(row0, ROWS_W)])


@jax.jit
def _mv(A, x):
    o_sc = _sc_mv(A, x)
    o_tc = _tc_mv(A, x, ROWS_SC, M - ROWS_SC)
    return jnp.concatenate([o_sc, o_tc])


def kernel(A, x):
    return _mv(A, x)
